# baseline (speedup 1.0000x reference)
import jax
import jax.numpy as jnp
from jax.experimental import pallas as pl
from jax.experimental.pallas import tpu as pltpu

_B = 4096
_N = 1000
_D = 16
_RB = 2048
_BIG = 1e30


def _ssm_body(u_ref, e_ref, idx_full_ref, idx_blk_ref, cnt_ref,
              out_ref, el_scr, lc_scr):
    b = pl.program_id(0)

    @pl.when(b == 0)
    def _prologue():
        idxf = idx_full_ref[:].astype(jnp.int16)
        ks = jax.lax.broadcasted_iota(jnp.int16, (_B, _N), 1)
        eq = (idxf == ks).astype(jnp.bfloat16)
        ones_b = jnp.ones((1, _B), dtype=jnp.bfloat16)
        c = jax.lax.dot_general(ones_b, eq, (((1,), (0,)), ((), ())),
                                preferred_element_type=jnp.float32)
        cnt_col = cnt_ref[:]
        cnt = jnp.transpose(cnt_col)
        total = jnp.sum(cnt)
        lq = jnp.log(jnp.where(cnt > 0.0, cnt, 1.0)) - jnp.log(total)
        lc = jnp.log(jnp.where(c > 0.5, c, 1.0))
        w = jnp.where(c > 0.5, lq - lc, _BIG)
        lc_scr[:] = lc
        el_scr[:, 0:_D] = e_ref[:]
        el_scr[:, _D:_D + 1] = jnp.transpose(-w)

    u = u_ref[:]
    up = jnp.concatenate(
        [u, jnp.ones((_RB, 1), dtype=jnp.float32)], axis=1)
    el = el_scr[:, 0:_D + 1]
    gp = jax.lax.dot_general(up, el, (((1,), (1,)), ((), ())),
                             preferred_element_type=jnp.float32)
    t = jnp.exp(gp).astype(jnp.bfloat16)
    ones_n = jnp.ones((_N, 1), dtype=jnp.bfloat16)
    s = jax.lax.dot_general(t, ones_n, (((1,), (0,)), ((), ())),
                            preferred_element_type=jnp.float32)
    lse = jnp.log(s)
    idxb = idx_blk_ref[:].astype(jnp.int16)
    p = (idxb == jax.lax.broadcasted_iota(jnp.int16, (_RB, _N), 1))
    y = gp - lc_scr[:]
    d = jnp.sum(jnp.where(p, y, 0.0), axis=1, keepdims=True)
    out_ref[:] = lse - d


@jax.jit
def kernel(item_embeddings, user_vec, item_count, item_idx):
    cnt_col = item_count.reshape(_N, 1)
    idx = item_idx.astype(jnp.int32)

    grid = (_B // _RB,)
    out = pl.pallas_call(
        _ssm_body,
        grid=grid,
        in_specs=[
            pl.BlockSpec((_RB, _D), lambda b: (b, 0)),
            pl.BlockSpec((_N, _D), lambda b: (0, 0)),
            pl.BlockSpec((_B, 1), lambda b: (0, 0)),
            pl.BlockSpec((_RB, 1), lambda b: (b, 0)),
            pl.BlockSpec((_N, 1), lambda b: (0, 0)),
        ],
        out_specs=pl.BlockSpec((_RB, 1), lambda b: (b, 0)),
        out_shape=jax.ShapeDtypeStruct((_B, 1), jnp.float32),
        scratch_shapes=[
            pltpu.VMEM((_N, 128), jnp.float32),
            pltpu.VMEM((1, _N), jnp.float32),
        ],
    )(user_vec, item_embeddings, idx, idx, cnt_col)
    return out


# speedup vs baseline: 1.1387x; 1.1387x over previous
import jax
import jax.numpy as jnp
from jax.experimental import pallas as pl
from jax.experimental.pallas import tpu as pltpu

_B = 4096
_N = 1000
_D = 16
_RB = 2048
_BIG = 1e30


def _ssm_body(u_ref, e_ref, idx_full_ref, cnt_ref,
              out_ref, el_scr, lc_scr):
    b = pl.program_id(0)

    @pl.when(b == 0)
    def _prologue():
        idxf = idx_full_ref[:]
        ks = jax.lax.broadcasted_iota(jnp.int32, (_B, _N), 1)
        eq = (idxf == ks).astype(jnp.bfloat16)
        ones_b = jnp.ones((1, _B), dtype=jnp.bfloat16)
        c = jax.lax.dot_general(ones_b, eq, (((1,), (0,)), ((), ())),
                                preferred_element_type=jnp.float32)
        cnt_col = cnt_ref[:]
        cnt = jnp.transpose(cnt_col)
        total = jnp.sum(cnt)
        lq = jnp.log(jnp.where(cnt > 0.0, cnt, 1.0)) - jnp.log(total)
        lc = jnp.log(jnp.where(c > 0.5, c, 1.0))
        w = jnp.where(c > 0.5, lq - lc, _BIG)
        lc_scr[:] = lc
        el_scr[:, 0:_D] = e_ref[:]
        el_scr[:, _D:_D + 1] = jnp.transpose(-w)

    u = u_ref[:]
    up = jnp.concatenate(
        [u, jnp.ones((_RB, 1), dtype=jnp.float32)], axis=1)
    el = el_scr[:, 0:_D + 1]
    gp = jax.lax.dot_general(up, el, (((1,), (1,)), ((), ())),
                             preferred_element_type=jnp.float32)
    t = jnp.exp(gp).astype(jnp.bfloat16)
    ones_n = jnp.ones((_N, 1), dtype=jnp.bfloat16)
    s = jax.lax.dot_general(t, ones_n, (((1,), (0,)), ((), ())),
                            preferred_element_type=jnp.float32)
    lse = jnp.log(s)
    idxb = idx_full_ref[pl.ds(b * _RB, _RB), :]
    p = (idxb == jax.lax.broadcasted_iota(jnp.int32, (_RB, _N), 1))
    y = gp - lc_scr[:]
    d = jnp.sum(jnp.where(p, y, 0.0), axis=1, keepdims=True)
    out_ref[:] = lse - d


@jax.jit
def kernel(item_embeddings, user_vec, item_count, item_idx):
    cnt_col = item_count.reshape(_N, 1)
    idx = item_idx.astype(jnp.int32)

    grid = (_B // _RB,)
    out = pl.pallas_call(
        _ssm_body,
        grid=grid,
        in_specs=[
            pl.BlockSpec((_RB, _D), lambda b: (b, 0)),
            pl.BlockSpec((_N, _D), lambda b: (0, 0)),
            pl.BlockSpec((_B, 1), lambda b: (0, 0)),
            pl.BlockSpec((_N, 1), lambda b: (0, 0)),
        ],
        out_specs=pl.BlockSpec((_RB, 1), lambda b: (b, 0)),
        out_shape=jax.ShapeDtypeStruct((_B, 1), jnp.float32),
        scratch_shapes=[
            pltpu.VMEM((_N, 128), jnp.float32),
            pltpu.VMEM((1, _N), jnp.float32),
        ],
    )(user_vec, item_embeddings, idx, cnt_col)
    return out
